# baseline (speedup 1.0000x reference)
import jax
import jax.numpy as jnp
from jax.experimental import pallas as pl
from jax.experimental.pallas import tpu as pltpu

N_NODES = 10
HIDDEN = 256


def _copy_body(src_hbm, out_hbm, sem):
    pltpu.make_async_copy(src_hbm, out_hbm, sem).start()
    pltpu.make_async_copy(src_hbm, out_hbm, sem).wait()


def kernel(node_feat, edge_feat, edge_list, intsc_feat_fc, messageNN, updateNN):
    del edge_feat, edge_list, intsc_feat_fc, messageNN, updateNN
    return pl.pallas_call(
        _copy_body,
        out_shape=jax.ShapeDtypeStruct((N_NODES, HIDDEN), jnp.float32),
        in_specs=[pl.BlockSpec(memory_space=pl.ANY)],
        out_specs=pl.BlockSpec(memory_space=pl.ANY),
        scratch_shapes=[pltpu.SemaphoreType.DMA],
    )(node_feat)


# speedup vs baseline: 1.1624x; 1.1624x over previous
import jax
import jax.numpy as jnp
from jax.experimental import pallas as pl

N_NODES = 10
HIDDEN = 256


def _copy_body(src_ref, out_ref):
    out_ref[...] = src_ref[...]


def kernel(node_feat, edge_feat, edge_list, intsc_feat_fc, messageNN, updateNN):
    del edge_feat, edge_list, intsc_feat_fc, messageNN, updateNN
    return pl.pallas_call(
        _copy_body,
        out_shape=jax.ShapeDtypeStruct((N_NODES, HIDDEN), jnp.float32),
    )(node_feat)
